# Initial kernel scaffold; baseline (speedup 1.0000x reference)
import jax
import jax.numpy as jnp
from jax.experimental import pallas as pl


def kernel(x, time_fra, frame_emb, pe):
    raise NotImplementedError("write your pallas kernel here")


# speedup vs baseline: 1.1680x; 1.1680x over previous
import functools

import jax
import jax.numpy as jnp
from jax import lax
from jax.experimental import pallas as pl
from jax.experimental.pallas import tpu as pltpu
from jax.experimental.pallas import tpu_sc as plsc

NUM_CORES = 2
NUM_SUBCORES = 16
NUM_WORKERS = NUM_CORES * NUM_SUBCORES

LANES = 16
CHUNK = 32


def _pe_add_kernel(n_rows: int, d: int):
    rows_per_w = n_rows // NUM_WORKERS
    n_chunks = rows_per_w // CHUNK
    mesh = plsc.VectorSubcoreMesh(core_axis_name="c", subcore_axis_name="s")

    @functools.partial(
        pl.kernel,
        mesh=mesh,
        out_type=jax.ShapeDtypeStruct((n_rows, d), jnp.float32),
        scratch_types=[
            pltpu.VMEM((CHUNK,), jnp.int32),
            pltpu.VMEM((CHUNK, d), jnp.float32),
            pltpu.VMEM((CHUNK, d), jnp.float32),
            pltpu.SemaphoreType.DMA,
        ],
    )
    def body(x_hbm, idx_hbm, pe_hbm, out_hbm, idx_v, pe_buf, x_buf, sem):
        wid = lax.axis_index("s") * NUM_CORES + lax.axis_index("c")
        base0 = wid * rows_per_w

        def chunk_body(i, carry):
            base = base0 + i * CHUNK
            pltpu.sync_copy(idx_hbm.at[pl.ds(base, CHUNK)], idx_v)
            gat = pltpu.async_copy(pe_hbm.at[idx_v], pe_buf, sem)
            pltpu.sync_copy(x_hbm.at[pl.ds(base, CHUNK)], x_buf)
            gat.wait()

            @plsc.parallel_loop(0, CHUNK)
            def row_body(r):
                for c in range(d // LANES):
                    sl = pl.ds(c * LANES, LANES)
                    plsc.addupdate(pe_buf.at[r, sl], x_buf[r, sl])

            pltpu.sync_copy(pe_buf, out_hbm.at[pl.ds(base, CHUNK)])
            return carry

        lax.fori_loop(0, n_chunks, chunk_body, 0)

    return body


def kernel(x, time_fra, frame_emb, pe):
    b, s, d = x.shape
    n = b * s
    xf = x.reshape(n, d)
    idx = time_fra.reshape(n).astype(jnp.int32)
    out = _pe_add_kernel(n, d)(xf, idx, pe)
    return out.reshape(b, s, d)


# speedup vs baseline: 1.8478x; 1.5820x over previous
import functools

import jax
import jax.numpy as jnp
from jax import lax
from jax.experimental import pallas as pl
from jax.experimental.pallas import tpu as pltpu
from jax.experimental.pallas import tpu_sc as plsc

NUM_CORES = 2
NUM_SUBCORES = 16
NUM_WORKERS = NUM_CORES * NUM_SUBCORES

LANES = 16
CHUNK = 16
NPE = 2
NX = 4


def _pe_add_kernel(n_rows: int, d: int):
    rows_per_w = n_rows // NUM_WORKERS
    n_chunks = rows_per_w // CHUNK
    assert n_chunks % NX == 0 and n_chunks >= 2 * NX
    mesh = plsc.VectorSubcoreMesh(core_axis_name="c", subcore_axis_name="s")

    @functools.partial(
        pl.kernel,
        mesh=mesh,
        out_type=jax.ShapeDtypeStruct((n_rows, d), jnp.float32),
        scratch_types=[
            [pltpu.VMEM((CHUNK,), jnp.int32) for _ in range(NPE)],
            [pltpu.VMEM((CHUNK, d), jnp.float32) for _ in range(NPE)],
            [pltpu.VMEM((CHUNK, d), jnp.float32) for _ in range(NX)],
            [pltpu.SemaphoreType.DMA for _ in range(NPE)],
            [pltpu.SemaphoreType.DMA for _ in range(NX)],
            [pltpu.SemaphoreType.DMA for _ in range(NX)],
        ],
    )
    def body(x_hbm, idx_hbm, pe_hbm, out_hbm,
             idx_v, pe_buf, x_buf, sem_g, sem_x, sem_o):
        wid = lax.axis_index("s") * NUM_CORES + lax.axis_index("c")
        base0 = wid * rows_per_w

        def issue_inputs(j, bp, bx):
            base = base0 + j * CHUNK
            pltpu.sync_copy(idx_hbm.at[pl.ds(base, CHUNK)], idx_v[bp])
            pltpu.async_copy(pe_hbm.at[idx_v[bp]], pe_buf[bp], sem_g[bp])
            pltpu.async_copy(x_hbm.at[pl.ds(base, CHUNK)], x_buf[bx], sem_x[bx])

        for b in range(NPE):
            issue_inputs(b, b, b)

        @pl.loop(0, n_chunks, step=NX)
        def chunk_group(g):
            for b in range(NX):
                bp = b % NPE
                j = g + b
                base = base0 + j * CHUNK
                pltpu.make_async_copy(pe_hbm.at[idx_v[bp]], pe_buf[bp],
                                      sem_g[bp]).wait()
                pltpu.make_async_copy(x_hbm.at[pl.ds(base, CHUNK)],
                                      x_buf[b], sem_x[b]).wait()

                @plsc.parallel_loop(0, CHUNK)
                def row_body(r):
                    for c in range(d // LANES):
                        sl = pl.ds(c * LANES, LANES)
                        plsc.addupdate(x_buf[b].at[r, sl], pe_buf[bp][r, sl])

                pltpu.async_copy(x_buf[b], out_hbm.at[pl.ds(base, CHUNK)],
                                 sem_o[b])

                b2 = (b + 2) % NX

                @pl.when(j >= 2)
                def _():
                    pltpu.make_async_copy(
                        x_buf[b2],
                        out_hbm.at[pl.ds(base - 2 * CHUNK, CHUNK)],
                        sem_o[b2]).wait()

                @pl.when(j + 2 < n_chunks)
                def _():
                    issue_inputs(j + 2, bp, b2)

        for j in (n_chunks - 2, n_chunks - 1):
            base = base0 + j * CHUNK
            pltpu.make_async_copy(x_buf[j % NX],
                                  out_hbm.at[pl.ds(base, CHUNK)],
                                  sem_o[j % NX]).wait()

    return body


def kernel(x, time_fra, frame_emb, pe):
    b, s, d = x.shape
    n = b * s
    xf = x.reshape(n, d)
    idx = time_fra.reshape(n).astype(jnp.int32)
    out = _pe_add_kernel(n, d)(xf, idx, pe)
    return out.reshape(b, s, d)
